# baseline (speedup 1.0000x reference)
import jax
import jax.numpy as jnp
from jax.experimental import pallas as pl
from jax.experimental.pallas import tpu as pltpu

P, N, S1, S2, K = 16, 1024, 256, 64, 32
R1, R2 = 0.15, 0.3
NEG = -1e9
F32 = jnp.float32
NINF = float("-inf")


def _fps_planes(p0, p1, p2, n, s):
    lane = jax.lax.broadcasted_iota(jnp.int32, (1, n), 1)
    slane = jax.lax.broadcasted_iota(jnp.int32, (1, s), 1)

    def body(i, st):
        dists, far, q0, q1, q2 = st
        m = (lane == far).astype(F32)
        c0 = jnp.sum(p0 * m, axis=1, keepdims=True)
        c1 = jnp.sum(p1 * m, axis=1, keepdims=True)
        c2 = jnp.sum(p2 * m, axis=1, keepdims=True)
        oh = (slane == i).astype(F32)
        q0 = q0 + c0 * oh
        q1 = q1 + c1 * oh
        q2 = q2 + c2 * oh
        d = (p0 - c0) ** 2 + (p1 - c1) ** 2 + (p2 - c2) ** 2
        dists = jnp.minimum(dists, d)
        rmax = jnp.max(dists, axis=1, keepdims=True)
        far = jnp.min(jnp.where(dists == rmax, lane, n), axis=1, keepdims=True)
        return dists, far, q0, q1, q2

    init = (
        jnp.full((P, n), jnp.inf, F32),
        jnp.zeros((P, 1), jnp.int32),
        jnp.zeros((P, s), F32),
        jnp.zeros((P, s), F32),
        jnp.zeros((P, s), F32),
    )
    _, _, q0, q1, q2 = jax.lax.fori_loop(0, s, body, init)
    return q0, q1, q2


def _prep_body(posT_ref, posnT_ref, q1T_ref, q2T_ref, vminT_ref, diff_ref):
    p0 = posT_ref[0]
    p1 = posT_ref[1]
    p2 = posT_ref[2]
    mn0 = jnp.min(p0, axis=1, keepdims=True)
    mn1 = jnp.min(p1, axis=1, keepdims=True)
    mn2 = jnp.min(p2, axis=1, keepdims=True)
    mx0 = jnp.max(p0, axis=1, keepdims=True)
    mx1 = jnp.max(p1, axis=1, keepdims=True)
    mx2 = jnp.max(p2, axis=1, keepdims=True)
    diff = jnp.maximum(jnp.maximum(mx0 - mn0, mx1 - mn1), mx2 - mn2)
    n0 = (p0 - mn0) / diff
    n1 = (p1 - mn1) / diff
    n2 = (p2 - mn2) / diff
    posnT_ref[0] = n0
    posnT_ref[1] = n1
    posnT_ref[2] = n2
    vminT_ref[0] = mn0
    vminT_ref[1] = mn1
    vminT_ref[2] = mn2
    diff_ref[...] = diff
    a0, a1, a2 = _fps_planes(n0, n1, n2, N, S1)
    q1T_ref[0] = a0
    q1T_ref[1] = a1
    q1T_ref[2] = a2
    b0, b1, b2 = _fps_planes(a0, a1, a2, S1, S2)
    q2T_ref[0] = b0
    q2T_ref[1] = b1
    q2T_ref[2] = b2


def _conv_body(posnT_ref, q1T_ref, q2T_ref, posnR_ref, q1R_ref, q2R_ref,
               W1a_ref, b1a_ref, W1b_ref, b1b_ref,
               W2a_ref, b2a_ref, W2b_ref, b2b_ref,
               W3a_ref, b3a_ref, W3b_ref, b3b_ref,
               g_ref, x2_ref, M1_ref, M2_ref):
    p0 = posnT_ref[0, 0]
    p1 = posnT_ref[1, 0]
    p2 = posnT_ref[2, 0]
    a0 = q1T_ref[0, 0]
    a1 = q1T_ref[1, 0]
    a2 = q1T_ref[2, 0]
    c1_0 = q1R_ref[0][:, 0:1]
    c1_1 = q1R_ref[0][:, 1:2]
    c1_2 = q1R_ref[0][:, 2:3]
    c2_0 = q2R_ref[0][:, 0:1]
    c2_1 = q2R_ref[0][:, 1:2]
    c2_2 = q2R_ref[0][:, 2:3]

    laneN = jax.lax.broadcasted_iota(jnp.int32, (1, N), 1)
    laneS1 = jax.lax.broadcasted_iota(jnp.int32, (1, S1), 1)

    d2 = (c1_0 - p0) ** 2 + (c1_1 - p1) ** 2 + (c1_2 - p2) ** 2
    inr1 = d2 <= R1 * R1
    work0 = jnp.where(inr1, -d2, NINF)
    nv1 = jnp.sum(inr1.astype(F32), axis=1, keepdims=True)

    def body1(t, work):
        rmax = jnp.max(work, axis=1, keepdims=True)
        first = jnp.min(jnp.where(work == rmax, laneN, N), axis=1,
                        keepdims=True)
        m = laneN == first
        M1_ref[pl.ds(t * S1, S1), :] = m.astype(F32)
        return jnp.where(m, NINF, work)

    jax.lax.fori_loop(0, K, body1, work0)

    ws0 = W1a_ref[0:1, :] + W1a_ref[3:4, :]
    ws1 = W1a_ref[1:2, :] + W1a_ref[4:5, :]
    ws2 = W1a_ref[2:3, :] + W1a_ref[5:6, :]
    hbase1 = (b1a_ref[...]
              - c1_0 * W1a_ref[3:4, :]
              - c1_1 * W1a_ref[4:5, :]
              - c1_2 * W1a_ref[5:6, :])
    wsum = jnp.concatenate([ws0, ws1, ws2], axis=0)
    psel = jnp.dot(M1_ref[...], posnR_ref[0],
                   preferred_element_type=F32)
    t1 = jnp.dot(psel, wsum, preferred_element_type=F32)
    h1 = jnp.maximum(t1.reshape(K, S1, 64) + hbase1[None], 0.0)
    msg1 = jnp.maximum(
        jnp.dot(h1.reshape(K * S1, 64), W1b_ref[...],
                preferred_element_type=F32) + b1b_ref[...], 0.0)
    slot = jax.lax.broadcasted_iota(jnp.int32, (K, 1), 0).reshape(K, 1, 1)
    val1 = slot.astype(F32) < nv1.reshape(1, S1, 1)
    x1 = jnp.max(jnp.where(val1, msg1.reshape(K, S1, 128), NEG),
                 axis=0)

    d2b = (c2_0 - a0) ** 2 + (c2_1 - a1) ** 2 + (c2_2 - a2) ** 2
    inr2 = d2b <= R2 * R2
    work0b = jnp.where(inr2, -d2b, NINF)
    nv2 = jnp.sum(inr2.astype(F32), axis=1, keepdims=True)

    def body2(t, work):
        rmax = jnp.max(work, axis=1, keepdims=True)
        first = jnp.min(jnp.where(work == rmax, laneS1, S1), axis=1,
                        keepdims=True)
        m = laneS1 == first
        M2_ref[pl.ds(t * S2, S2), :] = m.astype(F32)
        return jnp.where(m, NINF, work)

    jax.lax.fori_loop(0, K, body2, work0b)

    wp0 = W2a_ref[128:129, :]
    wp1 = W2a_ref[129:130, :]
    wp2 = W2a_ref[130:131, :]
    hbase2 = (b2a_ref[...]
              - c2_0 * wp0 - c2_1 * wp1 - c2_2 * wp2)
    cat = jnp.concatenate([x1, q1R_ref[0]], axis=1)
    gsel = jnp.dot(M2_ref[...], cat, preferred_element_type=F32)
    t2 = jnp.dot(gsel, W2a_ref[...], preferred_element_type=F32)
    h2 = jnp.maximum(t2.reshape(K, S2, 256) + hbase2[None], 0.0)
    msg2 = jnp.maximum(
        jnp.dot(h2.reshape(K * S2, 256), W2b_ref[...],
                preferred_element_type=F32) + b2b_ref[...], 0.0)
    val2 = slot.astype(F32) < nv2.reshape(1, S2, 1)
    x2 = jnp.max(jnp.where(val2, msg2.reshape(K, S2, 384), NEG),
                 axis=0)

    h3 = jnp.maximum(
        jnp.dot(x2, W3a_ref[0:384, :], preferred_element_type=F32)
        + c2_0 * W3a_ref[384:385, :]
        + c2_1 * W3a_ref[385:386, :]
        + c2_2 * W3a_ref[386:387, :]
        + b3a_ref[...], 0.0)
    out = jnp.maximum(
        jnp.dot(h3, W3b_ref[...], preferred_element_type=F32)
        + b3b_ref[...], 0.0)
    g_ref[0] = jnp.max(out, axis=0, keepdims=True)
    x2_ref[0] = x2


def kernel(pos, pi, W1a, b1a, W1b, b1b, W2a, b2a, W2b, b2b,
           W3a, b3a, W3b, b3b):
    del pi
    posT = pos.reshape(P, N, 3).transpose(2, 0, 1)

    posnT, q1T, q2T, vminT, diffo = pl.pallas_call(
        _prep_body,
        out_shape=(
            jax.ShapeDtypeStruct((3, P, N), F32),
            jax.ShapeDtypeStruct((3, P, S1), F32),
            jax.ShapeDtypeStruct((3, P, S2), F32),
            jax.ShapeDtypeStruct((3, P, 1), F32),
            jax.ShapeDtypeStruct((P, 1), F32),
        ),
    )(posT)

    posnR = posnT.transpose(1, 2, 0)
    q1R = q1T.transpose(1, 2, 0)
    q2R = q2T.transpose(1, 2, 0)
    posnT4 = posnT.reshape(3, P, 1, N)
    q1T4 = q1T.reshape(3, P, 1, S1)
    q2T4 = q2T.reshape(3, P, 1, S2)

    full = lambda s: pl.BlockSpec(s, lambda p: (0,) * len(s))

    g, x2o = pl.pallas_call(
        _conv_body,
        grid=(P,),
        in_specs=[
            pl.BlockSpec((3, 1, 1, N), lambda p: (0, p, 0, 0)),
            pl.BlockSpec((3, 1, 1, S1), lambda p: (0, p, 0, 0)),
            pl.BlockSpec((3, 1, 1, S2), lambda p: (0, p, 0, 0)),
            pl.BlockSpec((1, N, 3), lambda p: (p, 0, 0)),
            pl.BlockSpec((1, S1, 3), lambda p: (p, 0, 0)),
            pl.BlockSpec((1, S2, 3), lambda p: (p, 0, 0)),
            full((6, 64)), full((1, 64)),
            full((64, 128)), full((1, 128)),
            full((131, 256)), full((1, 256)),
            full((256, 384)), full((1, 384)),
            full((387, 512)), full((1, 512)),
            full((512, 768)), full((1, 768)),
        ],
        out_specs=(
            pl.BlockSpec((1, 1, 768), lambda p: (p, 0, 0)),
            pl.BlockSpec((1, S2, 384), lambda p: (p, 0, 0)),
        ),
        out_shape=(
            jax.ShapeDtypeStruct((P, 1, 768), F32),
            jax.ShapeDtypeStruct((P, S2, 384), F32),
        ),
        scratch_shapes=[
            pltpu.VMEM((K * S1, N), F32),
            pltpu.VMEM((K * S2, S1), F32),
        ],
    )(posnT4, q1T4, q2T4, posnR, q1R, q2R,
      W1a, b1a.reshape(1, -1), W1b, b1b.reshape(1, -1),
      W2a, b2a.reshape(1, -1), W2b, b2b.reshape(1, -1),
      W3a, b3a.reshape(1, -1), W3b, b3b.reshape(1, -1))

    g = g.reshape(P, 768)
    q2 = q2R.reshape(P * S2, 3)
    v_min = vminT[:, :, 0].T
    diff = diffo[:, 0]
    g_pos = jnp.zeros((P, 3), F32)
    g_batch = jnp.arange(P)
    p2_batch = jnp.repeat(jnp.arange(P), S2)
    return (g, g_pos, g_batch, x2o.reshape(P * S2, 384), q2,
            p2_batch, v_min, diff)


# speedup vs baseline: 1.0790x; 1.0790x over previous
import jax
import jax.numpy as jnp
from jax.experimental import pallas as pl
from jax.experimental.pallas import tpu as pltpu

P, N, S1, S2, K = 16, 1024, 256, 64, 32
R1, R2 = 0.15, 0.3
NEG = -1e9
F32 = jnp.float32
NINF = float("-inf")


def _fps_planes(p0, p1, p2, n, s):
    lane = jax.lax.broadcasted_iota(jnp.int32, (1, n), 1)
    slane = jax.lax.broadcasted_iota(jnp.int32, (1, s), 1)

    def body(i, st):
        dists, far, q0, q1, q2 = st
        m = (lane == far).astype(F32)
        c0 = jnp.sum(p0 * m, axis=1, keepdims=True)
        c1 = jnp.sum(p1 * m, axis=1, keepdims=True)
        c2 = jnp.sum(p2 * m, axis=1, keepdims=True)
        oh = (slane == i).astype(F32)
        q0 = q0 + c0 * oh
        q1 = q1 + c1 * oh
        q2 = q2 + c2 * oh
        d = (p0 - c0) ** 2 + (p1 - c1) ** 2 + (p2 - c2) ** 2
        dists = jnp.minimum(dists, d)
        rmax = jnp.max(dists, axis=1, keepdims=True)
        far = jnp.min(jnp.where(dists == rmax, lane, n), axis=1, keepdims=True)
        return dists, far, q0, q1, q2

    init = (
        jnp.full((P, n), jnp.inf, F32),
        jnp.zeros((P, 1), jnp.int32),
        jnp.zeros((P, s), F32),
        jnp.zeros((P, s), F32),
        jnp.zeros((P, s), F32),
    )
    _, _, q0, q1, q2 = jax.lax.fori_loop(0, s, body, init)
    return q0, q1, q2


def _prep_body(posT_ref, posnT_ref, q1T_ref, q2T_ref, vminT_ref, diff_ref):
    p0 = posT_ref[0]
    p1 = posT_ref[1]
    p2 = posT_ref[2]
    mn0 = jnp.min(p0, axis=1, keepdims=True)
    mn1 = jnp.min(p1, axis=1, keepdims=True)
    mn2 = jnp.min(p2, axis=1, keepdims=True)
    mx0 = jnp.max(p0, axis=1, keepdims=True)
    mx1 = jnp.max(p1, axis=1, keepdims=True)
    mx2 = jnp.max(p2, axis=1, keepdims=True)
    diff = jnp.maximum(jnp.maximum(mx0 - mn0, mx1 - mn1), mx2 - mn2)
    n0 = (p0 - mn0) / diff
    n1 = (p1 - mn1) / diff
    n2 = (p2 - mn2) / diff
    posnT_ref[0] = n0
    posnT_ref[1] = n1
    posnT_ref[2] = n2
    vminT_ref[0] = mn0
    vminT_ref[1] = mn1
    vminT_ref[2] = mn2
    diff_ref[...] = diff
    a0, a1, a2 = _fps_planes(n0, n1, n2, N, S1)
    q1T_ref[0] = a0
    q1T_ref[1] = a1
    q1T_ref[2] = a2
    b0, b1, b2 = _fps_planes(a0, a1, a2, S1, S2)
    q2T_ref[0] = b0
    q2T_ref[1] = b1
    q2T_ref[2] = b2


def _conv_body(posnT_ref, q1T_ref, q2T_ref, posnR_ref, q1R_ref, q2R_ref,
               W1a_ref, b1a_ref, W1b_ref, b1b_ref,
               W2a_ref, b2a_ref, W2b_ref, b2b_ref,
               W3a_ref, b3a_ref, W3b_ref, b3b_ref,
               g_ref, x2_ref, M1_ref, M2_ref):
    p0 = posnT_ref[0, 0]
    p1 = posnT_ref[1, 0]
    p2 = posnT_ref[2, 0]
    a0 = q1T_ref[0, 0]
    a1 = q1T_ref[1, 0]
    a2 = q1T_ref[2, 0]
    c1_0 = q1R_ref[0][:, 0:1]
    c1_1 = q1R_ref[0][:, 1:2]
    c1_2 = q1R_ref[0][:, 2:3]
    c2_0 = q2R_ref[0][:, 0:1]
    c2_1 = q2R_ref[0][:, 1:2]
    c2_2 = q2R_ref[0][:, 2:3]

    laneN = jax.lax.broadcasted_iota(jnp.int32, (1, N), 1)
    laneS1 = jax.lax.broadcasted_iota(jnp.int32, (1, S1), 1)

    d2 = (c1_0 - p0) ** 2 + (c1_1 - p1) ** 2 + (c1_2 - p2) ** 2
    inr1 = d2 <= R1 * R1
    work0 = jnp.where(inr1, -d2, NINF)
    nv1 = jnp.sum(inr1.astype(F32), axis=1, keepdims=True)

    def body1(t, work):
        rmax = jnp.max(work, axis=1, keepdims=True)
        first = jnp.min(jnp.where(work == rmax, laneN, N), axis=1,
                        keepdims=True)
        m = laneN == first
        M1_ref[pl.ds(2 * t * S1, S1), :] = m.astype(F32)
        work = jnp.where(m, NINF, work)
        rmax = jnp.max(work, axis=1, keepdims=True)
        first = jnp.min(jnp.where(work == rmax, laneN, N), axis=1,
                        keepdims=True)
        m = laneN == first
        M1_ref[pl.ds((2 * t + 1) * S1, S1), :] = m.astype(F32)
        return jnp.where(m, NINF, work)

    jax.lax.fori_loop(0, K // 2, body1, work0)

    ws0 = W1a_ref[0:1, :] + W1a_ref[3:4, :]
    ws1 = W1a_ref[1:2, :] + W1a_ref[4:5, :]
    ws2 = W1a_ref[2:3, :] + W1a_ref[5:6, :]
    hbase1 = (b1a_ref[...]
              - c1_0 * W1a_ref[3:4, :]
              - c1_1 * W1a_ref[4:5, :]
              - c1_2 * W1a_ref[5:6, :])
    wsum = jnp.concatenate([ws0, ws1, ws2], axis=0)
    psel = jnp.dot(M1_ref[...], posnR_ref[0],
                   preferred_element_type=F32)
    t1 = jnp.dot(psel, wsum, preferred_element_type=F32)
    h1 = jnp.maximum(t1.reshape(K, S1, 64) + hbase1[None], 0.0)
    msg1 = jnp.maximum(
        jnp.dot(h1.reshape(K * S1, 64), W1b_ref[...],
                preferred_element_type=F32) + b1b_ref[...], 0.0)
    slot = jax.lax.broadcasted_iota(jnp.int32, (K, 1), 0).reshape(K, 1, 1)
    val1 = slot.astype(F32) < nv1.reshape(1, S1, 1)
    x1 = jnp.max(jnp.where(val1, msg1.reshape(K, S1, 128), NEG),
                 axis=0)

    d2b = (c2_0 - a0) ** 2 + (c2_1 - a1) ** 2 + (c2_2 - a2) ** 2
    inr2 = d2b <= R2 * R2
    work0b = jnp.where(inr2, -d2b, NINF)
    nv2 = jnp.sum(inr2.astype(F32), axis=1, keepdims=True)

    def body2(t, work):
        rmax = jnp.max(work, axis=1, keepdims=True)
        first = jnp.min(jnp.where(work == rmax, laneS1, S1), axis=1,
                        keepdims=True)
        m = laneS1 == first
        M2_ref[pl.ds(2 * t * S2, S2), :] = m.astype(F32)
        work = jnp.where(m, NINF, work)
        rmax = jnp.max(work, axis=1, keepdims=True)
        first = jnp.min(jnp.where(work == rmax, laneS1, S1), axis=1,
                        keepdims=True)
        m = laneS1 == first
        M2_ref[pl.ds((2 * t + 1) * S2, S2), :] = m.astype(F32)
        return jnp.where(m, NINF, work)

    jax.lax.fori_loop(0, K // 2, body2, work0b)

    wp0 = W2a_ref[128:129, :]
    wp1 = W2a_ref[129:130, :]
    wp2 = W2a_ref[130:131, :]
    hbase2 = (b2a_ref[...]
              - c2_0 * wp0 - c2_1 * wp1 - c2_2 * wp2)
    cat = jnp.concatenate([x1, q1R_ref[0]], axis=1)
    gsel = jnp.dot(M2_ref[...], cat, preferred_element_type=F32)
    t2 = jnp.dot(gsel, W2a_ref[...], preferred_element_type=F32)
    h2 = jnp.maximum(t2.reshape(K, S2, 256) + hbase2[None], 0.0)
    msg2 = jnp.maximum(
        jnp.dot(h2.reshape(K * S2, 256), W2b_ref[...],
                preferred_element_type=F32) + b2b_ref[...], 0.0)
    val2 = slot.astype(F32) < nv2.reshape(1, S2, 1)
    x2 = jnp.max(jnp.where(val2, msg2.reshape(K, S2, 384), NEG),
                 axis=0)

    h3 = jnp.maximum(
        jnp.dot(x2, W3a_ref[0:384, :], preferred_element_type=F32)
        + c2_0 * W3a_ref[384:385, :]
        + c2_1 * W3a_ref[385:386, :]
        + c2_2 * W3a_ref[386:387, :]
        + b3a_ref[...], 0.0)
    out = jnp.maximum(
        jnp.dot(h3, W3b_ref[...], preferred_element_type=F32)
        + b3b_ref[...], 0.0)
    g_ref[0] = jnp.max(out, axis=0, keepdims=True)
    x2_ref[0] = x2


def kernel(pos, pi, W1a, b1a, W1b, b1b, W2a, b2a, W2b, b2b,
           W3a, b3a, W3b, b3b):
    del pi
    posT = pos.reshape(P, N, 3).transpose(2, 0, 1)

    posnT, q1T, q2T, vminT, diffo = pl.pallas_call(
        _prep_body,
        out_shape=(
            jax.ShapeDtypeStruct((3, P, N), F32),
            jax.ShapeDtypeStruct((3, P, S1), F32),
            jax.ShapeDtypeStruct((3, P, S2), F32),
            jax.ShapeDtypeStruct((3, P, 1), F32),
            jax.ShapeDtypeStruct((P, 1), F32),
        ),
    )(posT)

    posnR = posnT.transpose(1, 2, 0)
    q1R = q1T.transpose(1, 2, 0)
    q2R = q2T.transpose(1, 2, 0)
    posnT4 = posnT.reshape(3, P, 1, N)
    q1T4 = q1T.reshape(3, P, 1, S1)
    q2T4 = q2T.reshape(3, P, 1, S2)

    full = lambda s: pl.BlockSpec(s, lambda p: (0,) * len(s))

    g, x2o = pl.pallas_call(
        _conv_body,
        grid=(P,),
        in_specs=[
            pl.BlockSpec((3, 1, 1, N), lambda p: (0, p, 0, 0)),
            pl.BlockSpec((3, 1, 1, S1), lambda p: (0, p, 0, 0)),
            pl.BlockSpec((3, 1, 1, S2), lambda p: (0, p, 0, 0)),
            pl.BlockSpec((1, N, 3), lambda p: (p, 0, 0)),
            pl.BlockSpec((1, S1, 3), lambda p: (p, 0, 0)),
            pl.BlockSpec((1, S2, 3), lambda p: (p, 0, 0)),
            full((6, 64)), full((1, 64)),
            full((64, 128)), full((1, 128)),
            full((131, 256)), full((1, 256)),
            full((256, 384)), full((1, 384)),
            full((387, 512)), full((1, 512)),
            full((512, 768)), full((1, 768)),
        ],
        out_specs=(
            pl.BlockSpec((1, 1, 768), lambda p: (p, 0, 0)),
            pl.BlockSpec((1, S2, 384), lambda p: (p, 0, 0)),
        ),
        out_shape=(
            jax.ShapeDtypeStruct((P, 1, 768), F32),
            jax.ShapeDtypeStruct((P, S2, 384), F32),
        ),
        scratch_shapes=[
            pltpu.VMEM((K * S1, N), F32),
            pltpu.VMEM((K * S2, S1), F32),
        ],
    )(posnT4, q1T4, q2T4, posnR, q1R, q2R,
      W1a, b1a.reshape(1, -1), W1b, b1b.reshape(1, -1),
      W2a, b2a.reshape(1, -1), W2b, b2b.reshape(1, -1),
      W3a, b3a.reshape(1, -1), W3b, b3b.reshape(1, -1))

    g = g.reshape(P, 768)
    q2 = q2R.reshape(P * S2, 3)
    v_min = vminT[:, :, 0].T
    diff = diffo[:, 0]
    g_pos = jnp.zeros((P, 3), F32)
    g_batch = jnp.arange(P)
    p2_batch = jnp.repeat(jnp.arange(P), S2)
    return (g, g_pos, g_batch, x2o.reshape(P * S2, 384), q2,
            p2_batch, v_min, diff)


# speedup vs baseline: 1.1246x; 1.0423x over previous
import jax
import jax.numpy as jnp
from jax.experimental import pallas as pl
from jax.experimental.pallas import tpu as pltpu

P, N, S1, S2, K = 16, 1024, 256, 64, 32
R1, R2 = 0.15, 0.3
NEG = -1e9
F32 = jnp.float32
NINF = float("-inf")


def _fps_planes(p0, p1, p2, n, s):
    lane = jax.lax.broadcasted_iota(jnp.int32, (1, n), 1)
    slane = jax.lax.broadcasted_iota(jnp.int32, (1, s), 1)

    def body(i, st):
        dists, far, q0, q1, q2 = st
        m = (lane == far).astype(F32)
        c0 = jnp.sum(p0 * m, axis=1, keepdims=True)
        c1 = jnp.sum(p1 * m, axis=1, keepdims=True)
        c2 = jnp.sum(p2 * m, axis=1, keepdims=True)
        oh = (slane == i).astype(F32)
        q0 = q0 + c0 * oh
        q1 = q1 + c1 * oh
        q2 = q2 + c2 * oh
        d = (p0 - c0) ** 2 + (p1 - c1) ** 2 + (p2 - c2) ** 2
        dists = jnp.minimum(dists, d)
        rmax = jnp.max(dists, axis=1, keepdims=True)
        far = jnp.min(jnp.where(dists == rmax, lane, n), axis=1, keepdims=True)
        return dists, far, q0, q1, q2

    init = (
        jnp.full((P, n), jnp.inf, F32),
        jnp.zeros((P, 1), jnp.int32),
        jnp.zeros((P, s), F32),
        jnp.zeros((P, s), F32),
        jnp.zeros((P, s), F32),
    )
    _, _, q0, q1, q2 = jax.lax.fori_loop(0, s, body, init)
    return q0, q1, q2


def _prep_body(posT_ref, posnT_ref, q1T_ref, q2T_ref, vminT_ref, diff_ref):
    p0 = posT_ref[0]
    p1 = posT_ref[1]
    p2 = posT_ref[2]
    mn0 = jnp.min(p0, axis=1, keepdims=True)
    mn1 = jnp.min(p1, axis=1, keepdims=True)
    mn2 = jnp.min(p2, axis=1, keepdims=True)
    mx0 = jnp.max(p0, axis=1, keepdims=True)
    mx1 = jnp.max(p1, axis=1, keepdims=True)
    mx2 = jnp.max(p2, axis=1, keepdims=True)
    diff = jnp.maximum(jnp.maximum(mx0 - mn0, mx1 - mn1), mx2 - mn2)
    n0 = (p0 - mn0) / diff
    n1 = (p1 - mn1) / diff
    n2 = (p2 - mn2) / diff
    posnT_ref[0] = n0
    posnT_ref[1] = n1
    posnT_ref[2] = n2
    vminT_ref[0] = mn0
    vminT_ref[1] = mn1
    vminT_ref[2] = mn2
    diff_ref[...] = diff
    a0, a1, a2 = _fps_planes(n0, n1, n2, N, S1)
    q1T_ref[0] = a0
    q1T_ref[1] = a1
    q1T_ref[2] = a2
    b0, b1, b2 = _fps_planes(a0, a1, a2, S1, S2)
    q2T_ref[0] = b0
    q2T_ref[1] = b1
    q2T_ref[2] = b2


def _conv_body(posnT_ref, q1T_ref, q2T_ref, posnR_ref, q1R_ref, q2R_ref,
               W1a_ref, b1a_ref, W1b_ref, b1b_ref,
               W2a_ref, b2a_ref, W2b_ref, b2b_ref,
               W3a_ref, b3a_ref, W3b_ref, b3b_ref,
               g_ref, x2_ref, M1_ref, M2_ref):
    p0 = posnT_ref[0, 0]
    p1 = posnT_ref[1, 0]
    p2 = posnT_ref[2, 0]
    a0 = q1T_ref[0, 0]
    a1 = q1T_ref[1, 0]
    a2 = q1T_ref[2, 0]
    c1_0 = q1R_ref[0][:, 0:1]
    c1_1 = q1R_ref[0][:, 1:2]
    c1_2 = q1R_ref[0][:, 2:3]
    c2_0 = q2R_ref[0][:, 0:1]
    c2_1 = q2R_ref[0][:, 1:2]
    c2_2 = q2R_ref[0][:, 2:3]

    laneN = jax.lax.broadcasted_iota(jnp.int32, (1, N), 1)
    laneS1 = jax.lax.broadcasted_iota(jnp.int32, (1, S1), 1)

    d2 = (c1_0 - p0) ** 2 + (c1_1 - p1) ** 2 + (c1_2 - p2) ** 2
    inr1 = d2 <= R1 * R1
    work0 = jnp.where(inr1, -d2, NINF)
    nv1 = jnp.sum(inr1.astype(F32), axis=1, keepdims=True)

    def body1(t, work):
        for u in range(4):
            rmax = jnp.max(work, axis=1, keepdims=True)
            first = jnp.min(jnp.where(work == rmax, laneN, N), axis=1,
                            keepdims=True)
            m = laneN == first
            M1_ref[pl.ds((4 * t + u) * S1, S1), :] = m.astype(F32)
            work = jnp.where(m, NINF, work)
        return work

    jax.lax.fori_loop(0, K // 4, body1, work0)

    ws0 = W1a_ref[0:1, :] + W1a_ref[3:4, :]
    ws1 = W1a_ref[1:2, :] + W1a_ref[4:5, :]
    ws2 = W1a_ref[2:3, :] + W1a_ref[5:6, :]
    hbase1 = (b1a_ref[...]
              - c1_0 * W1a_ref[3:4, :]
              - c1_1 * W1a_ref[4:5, :]
              - c1_2 * W1a_ref[5:6, :])
    wsum = jnp.concatenate([ws0, ws1, ws2], axis=0)
    psel = jnp.dot(M1_ref[...], posnR_ref[0],
                   preferred_element_type=F32)
    t1 = jnp.dot(psel, wsum, preferred_element_type=F32)
    h1 = jnp.maximum(t1.reshape(K, S1, 64) + hbase1[None], 0.0)
    msg1 = jnp.maximum(
        jnp.dot(h1.reshape(K * S1, 64), W1b_ref[...],
                preferred_element_type=F32) + b1b_ref[...], 0.0)
    slot = jax.lax.broadcasted_iota(jnp.int32, (K, 1), 0).reshape(K, 1, 1)
    val1 = slot.astype(F32) < nv1.reshape(1, S1, 1)
    x1 = jnp.max(jnp.where(val1, msg1.reshape(K, S1, 128), NEG),
                 axis=0)

    d2b = (c2_0 - a0) ** 2 + (c2_1 - a1) ** 2 + (c2_2 - a2) ** 2
    inr2 = d2b <= R2 * R2
    work0b = jnp.where(inr2, -d2b, NINF)
    nv2 = jnp.sum(inr2.astype(F32), axis=1, keepdims=True)

    def body2(t, work):
        for u in range(4):
            rmax = jnp.max(work, axis=1, keepdims=True)
            first = jnp.min(jnp.where(work == rmax, laneS1, S1), axis=1,
                            keepdims=True)
            m = laneS1 == first
            M2_ref[pl.ds((4 * t + u) * S2, S2), :] = m.astype(F32)
            work = jnp.where(m, NINF, work)
        return work

    jax.lax.fori_loop(0, K // 4, body2, work0b)

    wp0 = W2a_ref[128:129, :]
    wp1 = W2a_ref[129:130, :]
    wp2 = W2a_ref[130:131, :]
    hbase2 = (b2a_ref[...]
              - c2_0 * wp0 - c2_1 * wp1 - c2_2 * wp2)
    cat = jnp.concatenate([x1, q1R_ref[0]], axis=1)
    gsel = jnp.dot(M2_ref[...], cat, preferred_element_type=F32)
    t2 = jnp.dot(gsel, W2a_ref[...], preferred_element_type=F32)
    h2 = jnp.maximum(t2.reshape(K, S2, 256) + hbase2[None], 0.0)
    msg2 = jnp.maximum(
        jnp.dot(h2.reshape(K * S2, 256), W2b_ref[...],
                preferred_element_type=F32) + b2b_ref[...], 0.0)
    val2 = slot.astype(F32) < nv2.reshape(1, S2, 1)
    x2 = jnp.max(jnp.where(val2, msg2.reshape(K, S2, 384), NEG),
                 axis=0)

    h3 = jnp.maximum(
        jnp.dot(x2, W3a_ref[0:384, :], preferred_element_type=F32)
        + c2_0 * W3a_ref[384:385, :]
        + c2_1 * W3a_ref[385:386, :]
        + c2_2 * W3a_ref[386:387, :]
        + b3a_ref[...], 0.0)
    out = jnp.maximum(
        jnp.dot(h3, W3b_ref[...], preferred_element_type=F32)
        + b3b_ref[...], 0.0)
    g_ref[0] = jnp.max(out, axis=0, keepdims=True)
    x2_ref[0] = x2


def kernel(pos, pi, W1a, b1a, W1b, b1b, W2a, b2a, W2b, b2b,
           W3a, b3a, W3b, b3b):
    del pi
    posT = pos.reshape(P, N, 3).transpose(2, 0, 1)

    posnT, q1T, q2T, vminT, diffo = pl.pallas_call(
        _prep_body,
        out_shape=(
            jax.ShapeDtypeStruct((3, P, N), F32),
            jax.ShapeDtypeStruct((3, P, S1), F32),
            jax.ShapeDtypeStruct((3, P, S2), F32),
            jax.ShapeDtypeStruct((3, P, 1), F32),
            jax.ShapeDtypeStruct((P, 1), F32),
        ),
    )(posT)

    posnR = posnT.transpose(1, 2, 0)
    q1R = q1T.transpose(1, 2, 0)
    q2R = q2T.transpose(1, 2, 0)
    posnT4 = posnT.reshape(3, P, 1, N)
    q1T4 = q1T.reshape(3, P, 1, S1)
    q2T4 = q2T.reshape(3, P, 1, S2)

    full = lambda s: pl.BlockSpec(s, lambda p: (0,) * len(s))

    g, x2o = pl.pallas_call(
        _conv_body,
        grid=(P,),
        in_specs=[
            pl.BlockSpec((3, 1, 1, N), lambda p: (0, p, 0, 0)),
            pl.BlockSpec((3, 1, 1, S1), lambda p: (0, p, 0, 0)),
            pl.BlockSpec((3, 1, 1, S2), lambda p: (0, p, 0, 0)),
            pl.BlockSpec((1, N, 3), lambda p: (p, 0, 0)),
            pl.BlockSpec((1, S1, 3), lambda p: (p, 0, 0)),
            pl.BlockSpec((1, S2, 3), lambda p: (p, 0, 0)),
            full((6, 64)), full((1, 64)),
            full((64, 128)), full((1, 128)),
            full((131, 256)), full((1, 256)),
            full((256, 384)), full((1, 384)),
            full((387, 512)), full((1, 512)),
            full((512, 768)), full((1, 768)),
        ],
        out_specs=(
            pl.BlockSpec((1, 1, 768), lambda p: (p, 0, 0)),
            pl.BlockSpec((1, S2, 384), lambda p: (p, 0, 0)),
        ),
        out_shape=(
            jax.ShapeDtypeStruct((P, 1, 768), F32),
            jax.ShapeDtypeStruct((P, S2, 384), F32),
        ),
        scratch_shapes=[
            pltpu.VMEM((K * S1, N), F32),
            pltpu.VMEM((K * S2, S1), F32),
        ],
    )(posnT4, q1T4, q2T4, posnR, q1R, q2R,
      W1a, b1a.reshape(1, -1), W1b, b1b.reshape(1, -1),
      W2a, b2a.reshape(1, -1), W2b, b2b.reshape(1, -1),
      W3a, b3a.reshape(1, -1), W3b, b3b.reshape(1, -1))

    g = g.reshape(P, 768)
    q2 = q2R.reshape(P * S2, 3)
    v_min = vminT[:, :, 0].T
    diff = diffo[:, 0]
    g_pos = jnp.zeros((P, 3), F32)
    g_batch = jnp.arange(P)
    p2_batch = jnp.repeat(jnp.arange(P), S2)
    return (g, g_pos, g_batch, x2o.reshape(P * S2, 384), q2,
            p2_batch, v_min, diff)


# speedup vs baseline: 1.1501x; 1.0227x over previous
import jax
import jax.numpy as jnp
from jax.experimental import pallas as pl
from jax.experimental.pallas import tpu as pltpu

P, N, S1, S2, K = 16, 1024, 256, 64, 32
R1, R2 = 0.15, 0.3
NEG = -1e9
F32 = jnp.float32
NINF = float("-inf")


def _fps_planes(p0, p1, p2, n, s):
    lane = jax.lax.broadcasted_iota(jnp.int32, (1, n), 1)
    slane = jax.lax.broadcasted_iota(jnp.int32, (1, s), 1)

    def body(i, st):
        dists, far, q0, q1, q2 = st
        m = (lane == far).astype(F32)
        c0 = jnp.sum(p0 * m, axis=1, keepdims=True)
        c1 = jnp.sum(p1 * m, axis=1, keepdims=True)
        c2 = jnp.sum(p2 * m, axis=1, keepdims=True)
        oh = (slane == i).astype(F32)
        q0 = q0 + c0 * oh
        q1 = q1 + c1 * oh
        q2 = q2 + c2 * oh
        d = (p0 - c0) ** 2 + (p1 - c1) ** 2 + (p2 - c2) ** 2
        dists = jnp.minimum(dists, d)
        rmax = jnp.max(dists, axis=1, keepdims=True)
        far = jnp.min(jnp.where(dists == rmax, lane, n), axis=1, keepdims=True)
        return dists, far, q0, q1, q2

    init = (
        jnp.full((P, n), jnp.inf, F32),
        jnp.zeros((P, 1), jnp.int32),
        jnp.zeros((P, s), F32),
        jnp.zeros((P, s), F32),
        jnp.zeros((P, s), F32),
    )
    _, _, q0, q1, q2 = jax.lax.fori_loop(0, s, body, init)
    return q0, q1, q2


def _prep_body(posT_ref, posnT_ref, q1T_ref, q2T_ref, vminT_ref, diff_ref):
    p0 = posT_ref[0]
    p1 = posT_ref[1]
    p2 = posT_ref[2]
    mn0 = jnp.min(p0, axis=1, keepdims=True)
    mn1 = jnp.min(p1, axis=1, keepdims=True)
    mn2 = jnp.min(p2, axis=1, keepdims=True)
    mx0 = jnp.max(p0, axis=1, keepdims=True)
    mx1 = jnp.max(p1, axis=1, keepdims=True)
    mx2 = jnp.max(p2, axis=1, keepdims=True)
    diff = jnp.maximum(jnp.maximum(mx0 - mn0, mx1 - mn1), mx2 - mn2)
    n0 = (p0 - mn0) / diff
    n1 = (p1 - mn1) / diff
    n2 = (p2 - mn2) / diff
    posnT_ref[0] = n0
    posnT_ref[1] = n1
    posnT_ref[2] = n2
    vminT_ref[0] = mn0
    vminT_ref[1] = mn1
    vminT_ref[2] = mn2
    diff_ref[...] = diff
    a0, a1, a2 = _fps_planes(n0, n1, n2, N, S1)
    q1T_ref[0] = a0
    q1T_ref[1] = a1
    q1T_ref[2] = a2
    b0, b1, b2 = _fps_planes(a0, a1, a2, S1, S2)
    q2T_ref[0] = b0
    q2T_ref[1] = b1
    q2T_ref[2] = b2


def _conv_body(posnT_ref, q1T_ref, q2T_ref, posnR_ref, q1R_ref, q2R_ref,
               W1a_ref, b1a_ref, W1b_ref, b1b_ref,
               W2a_ref, b2a_ref, W2b_ref, b2b_ref,
               W3a_ref, b3a_ref, W3b_ref, b3b_ref,
               g_ref, x2_ref, M1_ref, M2_ref):
    p0 = posnT_ref[0, 0]
    p1 = posnT_ref[1, 0]
    p2 = posnT_ref[2, 0]
    a0 = q1T_ref[0, 0]
    a1 = q1T_ref[1, 0]
    a2 = q1T_ref[2, 0]
    c1_0 = q1R_ref[0][:, 0:1]
    c1_1 = q1R_ref[0][:, 1:2]
    c1_2 = q1R_ref[0][:, 2:3]
    c2_0 = q2R_ref[0][:, 0:1]
    c2_1 = q2R_ref[0][:, 1:2]
    c2_2 = q2R_ref[0][:, 2:3]

    laneN = jax.lax.broadcasted_iota(jnp.int32, (1, N), 1)
    laneS1 = jax.lax.broadcasted_iota(jnp.int32, (1, S1), 1)

    d2 = (c1_0 - p0) ** 2 + (c1_1 - p1) ** 2 + (c1_2 - p2) ** 2
    inr1 = d2 <= R1 * R1
    work0 = jnp.where(inr1, -d2, NINF)
    nv1 = jnp.sum(inr1.astype(F32), axis=1, keepdims=True)

    def body1(t, work):
        for u in range(8):
            rmax = jnp.max(work, axis=1, keepdims=True)
            first = jnp.min(jnp.where(work == rmax, laneN, N), axis=1,
                            keepdims=True)
            m = laneN == first
            M1_ref[pl.ds((8 * t + u) * S1, S1), :] = m.astype(F32)
            work = jnp.where(m, NINF, work)
        return work

    jax.lax.fori_loop(0, K // 8, body1, work0)

    ws0 = W1a_ref[0:1, :] + W1a_ref[3:4, :]
    ws1 = W1a_ref[1:2, :] + W1a_ref[4:5, :]
    ws2 = W1a_ref[2:3, :] + W1a_ref[5:6, :]
    hbase1 = (b1a_ref[...]
              - c1_0 * W1a_ref[3:4, :]
              - c1_1 * W1a_ref[4:5, :]
              - c1_2 * W1a_ref[5:6, :])
    wsum = jnp.concatenate([ws0, ws1, ws2], axis=0)
    psel = jnp.dot(M1_ref[...], posnR_ref[0],
                   preferred_element_type=F32)
    t1 = jnp.dot(psel, wsum, preferred_element_type=F32)
    h1 = jnp.maximum(t1.reshape(K, S1, 64) + hbase1[None], 0.0)
    msg1 = jnp.maximum(
        jnp.dot(h1.reshape(K * S1, 64), W1b_ref[...],
                preferred_element_type=F32) + b1b_ref[...], 0.0)
    slot = jax.lax.broadcasted_iota(jnp.int32, (K, 1), 0).reshape(K, 1, 1)
    val1 = slot.astype(F32) < nv1.reshape(1, S1, 1)
    x1 = jnp.max(jnp.where(val1, msg1.reshape(K, S1, 128), NEG),
                 axis=0)

    d2b = (c2_0 - a0) ** 2 + (c2_1 - a1) ** 2 + (c2_2 - a2) ** 2
    inr2 = d2b <= R2 * R2
    work0b = jnp.where(inr2, -d2b, NINF)
    nv2 = jnp.sum(inr2.astype(F32), axis=1, keepdims=True)

    def body2(t, work):
        for u in range(8):
            rmax = jnp.max(work, axis=1, keepdims=True)
            first = jnp.min(jnp.where(work == rmax, laneS1, S1), axis=1,
                            keepdims=True)
            m = laneS1 == first
            M2_ref[pl.ds((8 * t + u) * S2, S2), :] = m.astype(F32)
            work = jnp.where(m, NINF, work)
        return work

    jax.lax.fori_loop(0, K // 8, body2, work0b)

    wp0 = W2a_ref[128:129, :]
    wp1 = W2a_ref[129:130, :]
    wp2 = W2a_ref[130:131, :]
    hbase2 = (b2a_ref[...]
              - c2_0 * wp0 - c2_1 * wp1 - c2_2 * wp2)
    cat = jnp.concatenate([x1, q1R_ref[0]], axis=1)
    gsel = jnp.dot(M2_ref[...], cat, preferred_element_type=F32)
    t2 = jnp.dot(gsel, W2a_ref[...], preferred_element_type=F32)
    h2 = jnp.maximum(t2.reshape(K, S2, 256) + hbase2[None], 0.0)
    msg2 = jnp.maximum(
        jnp.dot(h2.reshape(K * S2, 256), W2b_ref[...],
                preferred_element_type=F32) + b2b_ref[...], 0.0)
    val2 = slot.astype(F32) < nv2.reshape(1, S2, 1)
    x2 = jnp.max(jnp.where(val2, msg2.reshape(K, S2, 384), NEG),
                 axis=0)

    h3 = jnp.maximum(
        jnp.dot(x2, W3a_ref[0:384, :], preferred_element_type=F32)
        + c2_0 * W3a_ref[384:385, :]
        + c2_1 * W3a_ref[385:386, :]
        + c2_2 * W3a_ref[386:387, :]
        + b3a_ref[...], 0.0)
    out = jnp.maximum(
        jnp.dot(h3, W3b_ref[...], preferred_element_type=F32)
        + b3b_ref[...], 0.0)
    g_ref[0] = jnp.max(out, axis=0, keepdims=True)
    x2_ref[0] = x2


def kernel(pos, pi, W1a, b1a, W1b, b1b, W2a, b2a, W2b, b2b,
           W3a, b3a, W3b, b3b):
    del pi
    posT = pos.reshape(P, N, 3).transpose(2, 0, 1)

    posnT, q1T, q2T, vminT, diffo = pl.pallas_call(
        _prep_body,
        out_shape=(
            jax.ShapeDtypeStruct((3, P, N), F32),
            jax.ShapeDtypeStruct((3, P, S1), F32),
            jax.ShapeDtypeStruct((3, P, S2), F32),
            jax.ShapeDtypeStruct((3, P, 1), F32),
            jax.ShapeDtypeStruct((P, 1), F32),
        ),
    )(posT)

    posnR = posnT.transpose(1, 2, 0)
    q1R = q1T.transpose(1, 2, 0)
    q2R = q2T.transpose(1, 2, 0)
    posnT4 = posnT.reshape(3, P, 1, N)
    q1T4 = q1T.reshape(3, P, 1, S1)
    q2T4 = q2T.reshape(3, P, 1, S2)

    full = lambda s: pl.BlockSpec(s, lambda p: (0,) * len(s))

    g, x2o = pl.pallas_call(
        _conv_body,
        grid=(P,),
        in_specs=[
            pl.BlockSpec((3, 1, 1, N), lambda p: (0, p, 0, 0)),
            pl.BlockSpec((3, 1, 1, S1), lambda p: (0, p, 0, 0)),
            pl.BlockSpec((3, 1, 1, S2), lambda p: (0, p, 0, 0)),
            pl.BlockSpec((1, N, 3), lambda p: (p, 0, 0)),
            pl.BlockSpec((1, S1, 3), lambda p: (p, 0, 0)),
            pl.BlockSpec((1, S2, 3), lambda p: (p, 0, 0)),
            full((6, 64)), full((1, 64)),
            full((64, 128)), full((1, 128)),
            full((131, 256)), full((1, 256)),
            full((256, 384)), full((1, 384)),
            full((387, 512)), full((1, 512)),
            full((512, 768)), full((1, 768)),
        ],
        out_specs=(
            pl.BlockSpec((1, 1, 768), lambda p: (p, 0, 0)),
            pl.BlockSpec((1, S2, 384), lambda p: (p, 0, 0)),
        ),
        out_shape=(
            jax.ShapeDtypeStruct((P, 1, 768), F32),
            jax.ShapeDtypeStruct((P, S2, 384), F32),
        ),
        scratch_shapes=[
            pltpu.VMEM((K * S1, N), F32),
            pltpu.VMEM((K * S2, S1), F32),
        ],
    )(posnT4, q1T4, q2T4, posnR, q1R, q2R,
      W1a, b1a.reshape(1, -1), W1b, b1b.reshape(1, -1),
      W2a, b2a.reshape(1, -1), W2b, b2b.reshape(1, -1),
      W3a, b3a.reshape(1, -1), W3b, b3b.reshape(1, -1))

    g = g.reshape(P, 768)
    q2 = q2R.reshape(P * S2, 3)
    v_min = vminT[:, :, 0].T
    diff = diffo[:, 0]
    g_pos = jnp.zeros((P, 3), F32)
    g_batch = jnp.arange(P)
    p2_batch = jnp.repeat(jnp.arange(P), S2)
    return (g, g_pos, g_batch, x2o.reshape(P * S2, 384), q2,
            p2_batch, v_min, diff)
